# Initial kernel scaffold; baseline (speedup 1.0000x reference)
import jax
import jax.numpy as jnp
from jax.experimental import pallas as pl


def kernel(e1, e2, a12, angle_index12, aw1, ab1, aw2, ab2, ew1, eb1, ew2, eb2):
    raise NotImplementedError("write your pallas kernel here")


# speedup vs baseline: 1.4019x; 1.4019x over previous
import functools

import jax
import jax.numpy as jnp
from jax import lax
from jax.experimental import pallas as pl
from jax.experimental.pallas import tpu as pltpu
from jax.experimental.pallas import tpu_sc as plsc

E1 = 800000
E2 = 200000
A = 800000

NC = 2
NS = 16
NW = NC * NS

A_PAD = 819200
CH = 1024
G = CH // 128
PER_W = A_PAD // NW
NCH_B = PER_W // CH
PER_S = A_PAD // NS
NCH_D = PER_S // CH

E2H = E2 // 2
E2Q = E2H // 2
SROWS = E2Q + 16
STRIPE = 3128
ZLAST = SROWS - 15 * STRIPE
FLAST = E2Q - 15 * STRIPE

_SELU_SCALE = 1.0507009873554804934193349852946
_SELU_ALPHA = 1.6732632423543772848170429916717


def _selu(x):
    return _SELU_SCALE * jnp.where(x > 0, x, _SELU_ALPHA * (jnp.exp(x) - 1.0))


def _mm_body(x_ref, w_ref, o_ref):
    o_ref[...] = jnp.dot(x_ref[...], w_ref[...], preferred_element_type=jnp.float32)


def _rows_mm(x, w, blk):
    n, k = x.shape
    m = w.shape[1]
    return pl.pallas_call(
        _mm_body,
        grid=(n // blk,),
        in_specs=[pl.BlockSpec((blk, k), lambda i: (i, 0)),
                  pl.BlockSpec((k, m), lambda i: (0, 0))],
        out_specs=pl.BlockSpec((blk, m), lambda i: (i, 0)),
        out_shape=jax.ShapeDtypeStruct((n, m), jnp.float32),
    )(x, w)


def _angle_body(a12_ref, g_ref, wa_ref, b1_ref, w2_ref, b2_ref, o_ref):
    u = jnp.dot(a12_ref[...], wa_ref[...], preferred_element_type=jnp.float32)
    u = u + b1_ref[...] + g_ref[...]
    h = _selu(u)
    o_ref[...] = jnp.dot(h, w2_ref[...], preferred_element_type=jnp.float32) + b2_ref[...]


def _angle_mlp(a12p, g, wa, b1, w2, b2, blk):
    n = a12p.shape[0]
    return pl.pallas_call(
        _angle_body,
        grid=(n // blk,),
        in_specs=[pl.BlockSpec((blk, 16), lambda i: (i, 0)),
                  pl.BlockSpec((blk, 32), lambda i: (i, 0)),
                  pl.BlockSpec((16, 32), lambda i: (0, 0)),
                  pl.BlockSpec((1, 32), lambda i: (0, 0)),
                  pl.BlockSpec((32, 16), lambda i: (0, 0)),
                  pl.BlockSpec((1, 16), lambda i: (0, 0))],
        out_specs=pl.BlockSpec((blk, 16), lambda i: (i, 0)),
        out_shape=jax.ShapeDtypeStruct((n, 16), jnp.float32),
    )(a12p, g, wa, b1, w2, b2)


def _edge_body(s_ref, c_ref, e2_ref, w1a_ref, w1b_ref, b1_ref, w2_ref, b2_ref, o_ref):
    aggr = s_ref[...] / jnp.maximum(c_ref[...], 1.0)
    u = (jnp.dot(aggr, w1a_ref[...], preferred_element_type=jnp.float32)
         + jnp.dot(e2_ref[...], w1b_ref[...], preferred_element_type=jnp.float32)
         + b1_ref[...])
    h = _selu(u)
    o_ref[...] = jnp.dot(h, w2_ref[...], preferred_element_type=jnp.float32) + b2_ref[...]


def _edge_mlp(sums, cnt16, e2, w1a, w1b, b1, w2, b2, blk):
    n = e2.shape[0]
    return pl.pallas_call(
        _edge_body,
        grid=(n // blk,),
        in_specs=[pl.BlockSpec((blk, 16), lambda i: (i, 0)),
                  pl.BlockSpec((blk, 16), lambda i: (i, 0)),
                  pl.BlockSpec((blk, 64), lambda i: (i, 0)),
                  pl.BlockSpec((16, 32), lambda i: (0, 0)),
                  pl.BlockSpec((64, 32), lambda i: (0, 0)),
                  pl.BlockSpec((1, 32), lambda i: (0, 0)),
                  pl.BlockSpec((32, 64), lambda i: (0, 0)),
                  pl.BlockSpec((1, 64), lambda i: (0, 0))],
        out_specs=pl.BlockSpec((blk, 64), lambda i: (i, 0)),
        out_shape=jax.ShapeDtypeStruct((n, 64), jnp.float32),
    )(sums, cnt16, e2, w1a, w1b, b1, w2, b2)


_MESH = plsc.VectorSubcoreMesh(core_axis_name="c", subcore_axis_name="s")


@functools.partial(
    pl.kernel,
    out_type=jax.ShapeDtypeStruct((A_PAD, 32), jnp.float32),
    mesh=_MESH,
    scratch_types=[
        pltpu.VMEM((G, 128), jnp.int32),
        pltpu.VMEM((G, 128), jnp.int32),
        pltpu.VMEM((CH, 32), jnp.float32),
        pltpu.VMEM((CH, 32), jnp.float32),
        pltpu.SemaphoreType.DMA,
    ],
    compiler_params=pltpu.CompilerParams(use_tc_tiling_on_sc=False),
)
def _gather_kernel(row_hbm, col_hbm, p1_hbm, p2_hbm, g_hbm, rowv, colv, b1, b2, sem):
    c = lax.axis_index("c")
    s = lax.axis_index("s")
    wid = s * NC + c
    base = wid * PER_W

    def chunk(i, _):
        off = pl.multiple_of(base + i * CH, CH)
        offr = pl.multiple_of(off // 128, G)
        pltpu.sync_copy(row_hbm.at[pl.ds(offr, G)], rowv)
        pltpu.sync_copy(col_hbm.at[pl.ds(offr, G)], colv)
        cps = []
        for j in range(G):
            cps.append(pltpu.async_copy(
                p1_hbm.at[rowv.at[j]], b1.at[pl.ds(j * 128, 128)], sem))
            cps.append(pltpu.async_copy(
                p2_hbm.at[colv.at[j]], b2.at[pl.ds(j * 128, 128)], sem))
        for cp in cps:
            cp.wait()

        def add_row(r, _):
            b1[r, pl.ds(0, 16)] = b1[r, pl.ds(0, 16)] + b2[r, pl.ds(0, 16)]
            b1[r, pl.ds(16, 16)] = b1[r, pl.ds(16, 16)] + b2[r, pl.ds(16, 16)]
            return ()

        lax.fori_loop(0, CH, add_row, (), unroll=8)
        pltpu.sync_copy(b1, g_hbm.at[pl.ds(off, CH)])
        return ()

    lax.fori_loop(0, NCH_B, chunk, ())


@functools.partial(
    pl.kernel,
    out_type=(jax.ShapeDtypeStruct((E2, 16), jnp.float32),
              jax.ShapeDtypeStruct((E2, 16), jnp.float32)),
    mesh=_MESH,
    scratch_types=[
        pltpu.VMEM((G, 128), jnp.int32),
        pltpu.VMEM((G, 128), jnp.int32),
        pltpu.VMEM((CH, 16), jnp.float32),
        pltpu.VMEM((CH, 16), jnp.float32),
        pltpu.VMEM_SHARED((SROWS, 16), jnp.float32),
    ],
    compiler_params=pltpu.CompilerParams(use_tc_tiling_on_sc=False),
)
def _scatter_kernel(col_hbm, a_hbm, sums_hbm, cnt_hbm, colv, sidxv, abuf, cbuf, S):
    c = lax.axis_index("c")
    s = lax.axis_index("s")
    base = s * PER_S

    def fill_cbuf(val):
        def fb(i, _):
            cbuf[i, pl.ds(0, 16)] = jnp.full((16,), val, jnp.float32)
            return ()
        lax.fori_loop(0, CH, fb, (), unroll=8)

    def zero_stripe(r0, total):
        done = 0
        while done < total:
            nrows = min(1024, total - done)
            pltpu.sync_copy(cbuf.at[pl.ds(0, nrows)],
                            S.at[pl.ds(pl.multiple_of(r0 + done, 8), nrows)])
            done += nrows

    def zero_s():
        @pl.when(s < 15)
        def _():
            zero_stripe(s * STRIPE, STRIPE)

        @pl.when(s == 15)
        def _():
            zero_stripe(15 * STRIPE, ZLAST)

    def compute_idx(off, seg0):
        def vb(j, _):
            jj = j // 8
            kk = j - jj * 8
            cv = colv[jj, pl.ds(kk * 16, 16)]
            gidx = off + j * 16 + lax.iota(jnp.int32, 16)
            lcol = cv - seg0
            valid = (lcol >= 0) & (lcol < E2Q) & (gidx < A)
            sidxv[jj, pl.ds(kk * 16, 16)] = jnp.where(valid, lcol, E2Q)
            return ()
        lax.fori_loop(0, CH // 16, vb, (), unroll=4)

    def pass_loop(with_a, seg0):
        def chunk(i, _):
            off = pl.multiple_of(base + i * CH, CH)
            offr = pl.multiple_of(off // 128, G)
            pltpu.sync_copy(col_hbm.at[pl.ds(offr, G)], colv)
            if with_a:
                pltpu.sync_copy(a_hbm.at[pl.ds(off, CH)], abuf)
            compute_idx(off, seg0)
            src = abuf if with_a else cbuf
            for j in range(G):
                pltpu.sync_copy(src.at[pl.ds(j * 128, 128)],
                                S.at[sidxv.at[j]], add=True)
            return ()
        lax.fori_loop(0, NCH_D, chunk, ())

    def flush(dst_hbm, seg0):
        @pl.when(s < 15)
        def _():
            r0 = pl.multiple_of(s * STRIPE, 8)
            pltpu.sync_copy(S.at[pl.ds(r0, STRIPE)],
                            dst_hbm.at[pl.ds(pl.multiple_of(seg0 + r0, 8), STRIPE)])

        @pl.when(s == 15)
        def _():
            r0 = 15 * STRIPE
            pltpu.sync_copy(S.at[pl.ds(r0, FLAST)],
                            dst_hbm.at[pl.ds(pl.multiple_of(seg0 + r0, 8), FLAST)])

    for p in range(2):
        seg0 = c * E2H + p * E2Q
        fill_cbuf(0.0)
        zero_s()
        plsc.subcore_barrier()
        pass_loop(True, seg0)
        plsc.subcore_barrier()
        flush(sums_hbm, seg0)
        plsc.subcore_barrier()
        zero_s()
        fill_cbuf(1.0)
        plsc.subcore_barrier()
        pass_loop(False, seg0)
        plsc.subcore_barrier()
        flush(cnt_hbm, seg0)
        plsc.subcore_barrier()


def kernel(e1, e2, a12, angle_index12, aw1, ab1, aw2, ab2, ew1, eb1, ew2, eb2):
    w_a = aw1[:16]
    w_e1 = aw1[16:80]
    w_e2 = aw1[80:144]

    p1 = _rows_mm(e1, w_e1, 4000)
    p2 = _rows_mm(e2, w_e2, 4000)

    pad = A_PAD - A
    row2 = jnp.pad(angle_index12[0], (0, pad)).reshape(A_PAD // 128, 128)
    col2 = jnp.pad(angle_index12[1], (0, pad)).reshape(A_PAD // 128, 128)

    g = _gather_kernel(row2, col2, p1, p2)

    a12p = jnp.pad(a12, ((0, pad), (0, 0)))
    a = _angle_mlp(a12p, g, w_a, ab1.reshape(1, 32), aw2, ab2.reshape(1, 16), 4096)

    sums, cnt16 = _scatter_kernel(col2, a)

    out = _edge_mlp(sums, cnt16, e2, ew1[:16], ew1[16:], eb1.reshape(1, 32),
                    ew2, eb2.reshape(1, 64), 2000)
    return out


# speedup vs baseline: 2.1544x; 1.5368x over previous
import functools

import jax
import jax.numpy as jnp
from jax import lax
from jax.experimental import pallas as pl
from jax.experimental.pallas import tpu as pltpu
from jax.experimental.pallas import tpu_sc as plsc

E1 = 800000
E2 = 200000
A = 800000

NC = 2
NS = 16
NW = NC * NS

A_PAD = 819200
CH = 1024
G = CH // 128
PER_W = A_PAD // NW
CHB = 512
GB = CHB // 128
NCH_B = PER_W // CHB
PER_S = A_PAD // NS
NCH_D = PER_S // CH

E2H = E2 // 2
SROWS = E2H + 16
STRIPE = 6256
ZLAST = SROWS - 15 * STRIPE
FLAST = E2H - 15 * STRIPE
CHD = 512
GD = CHD // 128
NCHD = PER_S // CHD

_SELU_SCALE = 1.0507009873554804934193349852946
_SELU_ALPHA = 1.6732632423543772848170429916717


def _selu(x):
    return _SELU_SCALE * jnp.where(x > 0, x, _SELU_ALPHA * (jnp.exp(x) - 1.0))


def _mm_body(x_ref, w_ref, o_ref):
    o_ref[...] = jnp.dot(x_ref[...], w_ref[...], preferred_element_type=jnp.float32)


def _rows_mm(x, w, blk):
    n, k = x.shape
    m = w.shape[1]
    return pl.pallas_call(
        _mm_body,
        grid=(n // blk,),
        in_specs=[pl.BlockSpec((blk, k), lambda i: (i, 0)),
                  pl.BlockSpec((k, m), lambda i: (0, 0))],
        out_specs=pl.BlockSpec((blk, m), lambda i: (i, 0)),
        out_shape=jax.ShapeDtypeStruct((n, m), jnp.float32),
    )(x, w)


def _angle_body(a12_ref, g_ref, wa_ref, b1_ref, w2_ref, b2_ref, o_ref):
    u = jnp.dot(a12_ref[...], wa_ref[...], preferred_element_type=jnp.float32)
    u = u + b1_ref[...] + g_ref[...]
    h = _selu(u)
    o_ref[...] = jnp.dot(h, w2_ref[...], preferred_element_type=jnp.float32) + b2_ref[...]


def _angle_mlp(a12, g, wa, b1, w2, b2, blk):
    n = a12.shape[0]
    return pl.pallas_call(
        _angle_body,
        grid=(n // blk,),
        in_specs=[pl.BlockSpec((blk, 16), lambda i: (i, 0)),
                  pl.BlockSpec((blk, 32), lambda i: (i, 0)),
                  pl.BlockSpec((16, 32), lambda i: (0, 0)),
                  pl.BlockSpec((1, 32), lambda i: (0, 0)),
                  pl.BlockSpec((32, 16), lambda i: (0, 0)),
                  pl.BlockSpec((1, 16), lambda i: (0, 0))],
        out_specs=pl.BlockSpec((blk, 16), lambda i: (i, 0)),
        out_shape=jax.ShapeDtypeStruct((A_PAD, 16), jnp.float32),
    )(a12, g, wa, b1, w2, b2)


def _edge_body(s_ref, c_ref, e2_ref, w1a_ref, w1b_ref, b1_ref, w2_ref, b2_ref, o_ref):
    aggr = s_ref[...] / jnp.maximum(c_ref[...], 1.0)
    u = (jnp.dot(aggr, w1a_ref[...], preferred_element_type=jnp.float32)
         + jnp.dot(e2_ref[...], w1b_ref[...], preferred_element_type=jnp.float32)
         + b1_ref[...])
    h = _selu(u)
    o_ref[...] = jnp.dot(h, w2_ref[...], preferred_element_type=jnp.float32) + b2_ref[...]


def _edge_mlp(sums, cnt16, e2, w1a, w1b, b1, w2, b2, blk):
    n = e2.shape[0]
    return pl.pallas_call(
        _edge_body,
        grid=(n // blk,),
        in_specs=[pl.BlockSpec((blk, 16), lambda i: (i, 0)),
                  pl.BlockSpec((blk, 16), lambda i: (i, 0)),
                  pl.BlockSpec((blk, 64), lambda i: (i, 0)),
                  pl.BlockSpec((16, 32), lambda i: (0, 0)),
                  pl.BlockSpec((64, 32), lambda i: (0, 0)),
                  pl.BlockSpec((1, 32), lambda i: (0, 0)),
                  pl.BlockSpec((32, 64), lambda i: (0, 0)),
                  pl.BlockSpec((1, 64), lambda i: (0, 0))],
        out_specs=pl.BlockSpec((blk, 64), lambda i: (i, 0)),
        out_shape=jax.ShapeDtypeStruct((n, 64), jnp.float32),
    )(sums, cnt16, e2, w1a, w1b, b1, w2, b2)


_MESH = plsc.VectorSubcoreMesh(core_axis_name="c", subcore_axis_name="s")


@functools.partial(
    pl.kernel,
    out_type=jax.ShapeDtypeStruct((A_PAD, 32), jnp.float32),
    mesh=_MESH,
    scratch_types=[
        pltpu.VMEM((2, GB, 128), jnp.int32),
        pltpu.VMEM((2, GB, 128), jnp.int32),
        pltpu.VMEM((2, CHB, 32), jnp.float32),
        pltpu.VMEM((2, CHB, 32), jnp.float32),
        pltpu.VMEM((2, CHB, 32), jnp.float32),
        pltpu.SemaphoreType.DMA((2,)),
        pltpu.SemaphoreType.DMA((2,)),
    ],
    compiler_params=pltpu.CompilerParams(use_tc_tiling_on_sc=False),
)
def _gather_kernel(row_hbm, col_hbm, p1_hbm, p2_hbm, g_hbm, rowv, colv, b1, b2,
                   wbuf, gsem, wsem):
    c = lax.axis_index("c")
    s = lax.axis_index("s")
    wid = s * NC + c
    base = wid * PER_W

    def fire(i, p):
        off = pl.multiple_of(base + i * CHB, CHB)
        offr = pl.multiple_of(off // 128, GB)
        pltpu.sync_copy(row_hbm.at[pl.ds(offr, GB)], rowv.at[p])
        pltpu.sync_copy(col_hbm.at[pl.ds(offr, GB)], colv.at[p])
        for j in range(GB):
            pltpu.async_copy(p1_hbm.at[rowv.at[p, j]],
                             b1.at[p, pl.ds(j * 128, 128)], gsem.at[p])
            pltpu.async_copy(p2_hbm.at[colv.at[p, j]],
                             b2.at[p, pl.ds(j * 128, 128)], gsem.at[p])

    def drain_gathers(p):
        pltpu.make_async_copy(g_hbm.at[pl.ds(0, CHB)], b1.at[p], gsem.at[p]).wait()
        pltpu.make_async_copy(g_hbm.at[pl.ds(0, CHB)], b2.at[p], gsem.at[p]).wait()

    def drain_write(p):
        pltpu.make_async_copy(g_hbm.at[pl.ds(0, CHB)], wbuf.at[p], wsem.at[p]).wait()

    def consume(i, p):
        off = pl.multiple_of(base + i * CHB, CHB)
        drain_gathers(p)

        def add_row(r, _):
            wbuf[p, r, pl.ds(0, 16)] = b1[p, r, pl.ds(0, 16)] + b2[p, r, pl.ds(0, 16)]
            wbuf[p, r, pl.ds(16, 16)] = b1[p, r, pl.ds(16, 16)] + b2[p, r, pl.ds(16, 16)]
            return ()

        lax.fori_loop(0, CHB, add_row, (), unroll=8)
        pltpu.async_copy(wbuf.at[p], g_hbm.at[pl.ds(off, CHB)], wsem.at[p])

    fire(0, 0)

    def pair(i2, _):
        i0 = i2 * 2
        fire(i0 + 1, 1)

        @pl.when(i2 > 0)
        def _():
            drain_write(0)
        consume(i0, 0)

        @pl.when(i2 < NCH_B // 2 - 1)
        def _():
            fire(i0 + 2, 0)

        @pl.when(i2 > 0)
        def _():
            drain_write(1)
        consume(i0 + 1, 1)
        return ()

    lax.fori_loop(0, NCH_B // 2, pair, ())
    drain_write(0)
    drain_write(1)


@functools.partial(
    pl.kernel,
    out_type=(jax.ShapeDtypeStruct((E2, 16), jnp.float32),
              jax.ShapeDtypeStruct((E2, 16), jnp.float32)),
    mesh=_MESH,
    scratch_types=[
        pltpu.VMEM((2, GD, 128), jnp.int32),
        pltpu.VMEM((GD, 128), jnp.int32),
        pltpu.VMEM((2, CHD, 16), jnp.float32),
        pltpu.VMEM((128, 16), jnp.float32),
        pltpu.VMEM_SHARED((SROWS, 16), jnp.float32),
        pltpu.SemaphoreType.DMA((2,)),
    ],
    compiler_params=pltpu.CompilerParams(use_tc_tiling_on_sc=False),
)
def _scatter_kernel(col_hbm, a_hbm, sums_hbm, cnt_hbm, colv, sidxv, abuf, cbuf,
                    S, lsem):
    c = lax.axis_index("c")
    s = lax.axis_index("s")
    base = s * PER_S
    seg0 = c * E2H

    def fill_cbuf(val):
        def fb(i, _):
            cbuf[i, pl.ds(0, 16)] = jnp.full((16,), val, jnp.float32)
            return ()
        lax.fori_loop(0, 128, fb, (), unroll=8)

    def zero_stripe(r0, total):
        done = 0
        while done < total:
            nrows = min(128, total - done)
            pltpu.sync_copy(cbuf.at[pl.ds(0, nrows)],
                            S.at[pl.ds(pl.multiple_of(r0 + done, 8), nrows)])
            done += nrows

    def zero_s():
        @pl.when(s < 15)
        def _():
            zero_stripe(s * STRIPE, STRIPE)

        @pl.when(s == 15)
        def _():
            zero_stripe(15 * STRIPE, ZLAST)

    def fire(i, p, with_a):
        off = pl.multiple_of(base + i * CHD, CHD)
        offr = pl.multiple_of(off // 128, GD)
        pltpu.async_copy(col_hbm.at[pl.ds(offr, GD)], colv.at[p], lsem.at[p])
        if with_a:
            pltpu.async_copy(a_hbm.at[pl.ds(off, CHD)], abuf.at[p], lsem.at[p])

    def drain_load(p, with_a):
        pltpu.make_async_copy(col_hbm.at[pl.ds(0, GD)], colv.at[p],
                              lsem.at[p]).wait()
        if with_a:
            pltpu.make_async_copy(a_hbm.at[pl.ds(0, CHD)], abuf.at[p],
                                  lsem.at[p]).wait()

    def consume(i, p, with_a):
        off = pl.multiple_of(base + i * CHD, CHD)
        drain_load(p, with_a)

        def vb(j, _):
            jj = j // 8
            kk = j - jj * 8
            cv = colv[p, jj, pl.ds(kk * 16, 16)]
            gidx = off + j * 16 + lax.iota(jnp.int32, 16)
            lcol = cv - seg0
            valid = (lcol >= 0) & (lcol < E2H) & (gidx < A)
            sidxv[jj, pl.ds(kk * 16, 16)] = jnp.where(valid, lcol, E2H)
            return ()

        lax.fori_loop(0, CHD // 16, vb, (), unroll=4)
        for j in range(GD):
            src = abuf.at[p, pl.ds(j * 128, 128)] if with_a else cbuf
            pltpu.sync_copy(src, S.at[sidxv.at[j]], add=True)

    def sweep(with_a):
        fire(0, 0, with_a)

        def pair(i2, _):
            i0 = i2 * 2
            fire(i0 + 1, 1, with_a)
            consume(i0, 0, with_a)

            @pl.when(i2 < NCHD // 2 - 1)
            def _():
                fire(i0 + 2, 0, with_a)

            consume(i0 + 1, 1, with_a)
            return ()

        lax.fori_loop(0, NCHD // 2, pair, ())

    def flush(dst_hbm):
        @pl.when(s < 15)
        def _():
            r0 = pl.multiple_of(s * STRIPE, 8)
            pltpu.sync_copy(S.at[pl.ds(r0, STRIPE)],
                            dst_hbm.at[pl.ds(pl.multiple_of(seg0 + r0, 8), STRIPE)])

        @pl.when(s == 15)
        def _():
            r0 = 15 * STRIPE
            pltpu.sync_copy(S.at[pl.ds(r0, FLAST)],
                            dst_hbm.at[pl.ds(pl.multiple_of(seg0 + r0, 8), FLAST)])

    fill_cbuf(0.0)
    zero_s()
    plsc.subcore_barrier()
    sweep(True)
    plsc.subcore_barrier()
    flush(sums_hbm)
    plsc.subcore_barrier()
    zero_s()
    fill_cbuf(1.0)
    plsc.subcore_barrier()
    sweep(False)
    plsc.subcore_barrier()
    flush(cnt_hbm)


def kernel(e1, e2, a12, angle_index12, aw1, ab1, aw2, ab2, ew1, eb1, ew2, eb2):
    w_a = aw1[:16]
    w_e1 = aw1[16:80]
    w_e2 = aw1[80:144]

    p1 = _rows_mm(e1, w_e1, 4000)
    p2 = _rows_mm(e2, w_e2, 4000)

    pad = A_PAD - A
    row2 = jnp.pad(angle_index12[0], (0, pad)).reshape(A_PAD // 128, 128)
    col2 = jnp.pad(angle_index12[1], (0, pad)).reshape(A_PAD // 128, 128)

    g = _gather_kernel(row2, col2, p1, p2)

    a = _angle_mlp(a12, g, w_a, ab1.reshape(1, 32), aw2, ab2.reshape(1, 16), 4000)

    sums, cnt16 = _scatter_kernel(col2, a)

    out = _edge_mlp(sums, cnt16, e2, ew1[:16], ew1[16:], eb1.reshape(1, 32),
                    ew2, eb2.reshape(1, 64), 2000)
    return out


# speedup vs baseline: 2.2539x; 1.0462x over previous
import functools

import jax
import jax.numpy as jnp
from jax import lax
from jax.experimental import pallas as pl
from jax.experimental.pallas import tpu as pltpu
from jax.experimental.pallas import tpu_sc as plsc

E1 = 800000
E2 = 200000
A = 800000

NC = 2
NS = 16
NW = NC * NS

A_PAD = 819200
CH = 1024
G = CH // 128
PER_W = A_PAD // NW
CHB = 512
GB = CHB // 128
NCH_B = PER_W // CHB
PER_S = A_PAD // NS
NCH_D = PER_S // CH

E2H = E2 // 2
SROWS = E2H + 16
STRIPE = 6256
ZLAST = SROWS - 15 * STRIPE
FLAST = E2H - 15 * STRIPE
CHD = 512
GD = CHD // 128
NCHD = PER_S // CHD

_SELU_SCALE = 1.0507009873554804934193349852946
_SELU_ALPHA = 1.6732632423543772848170429916717


def _selu(x):
    return _SELU_SCALE * jnp.where(x > 0, x, _SELU_ALPHA * (jnp.exp(x) - 1.0))


def _mm_body(x_ref, w_ref, o_ref):
    r = jnp.dot(x_ref[...], w_ref[...], preferred_element_type=jnp.float32)
    o_ref[...] = r.astype(o_ref.dtype)


def _rows_mm(x, w, blk, out_dtype=jnp.float32):
    n, k = x.shape
    m = w.shape[1]
    return pl.pallas_call(
        _mm_body,
        grid=(n // blk,),
        in_specs=[pl.BlockSpec((blk, k), lambda i: (i, 0)),
                  pl.BlockSpec((k, m), lambda i: (0, 0))],
        out_specs=pl.BlockSpec((blk, m), lambda i: (i, 0)),
        out_shape=jax.ShapeDtypeStruct((n, m), out_dtype),
    )(x, w)


def _angle_body(a12_ref, g_ref, wa_ref, b1_ref, w2_ref, b2_ref, o_ref):
    u = jnp.dot(a12_ref[...], wa_ref[...], preferred_element_type=jnp.float32)
    u = u + b1_ref[...] + g_ref[...].astype(jnp.float32)
    h = _selu(u)
    o_ref[...] = jnp.dot(h, w2_ref[...], preferred_element_type=jnp.float32) + b2_ref[...]


def _angle_mlp(a12, g, wa, b1, w2, b2, blk):
    n = a12.shape[0]
    return pl.pallas_call(
        _angle_body,
        grid=(n // blk,),
        in_specs=[pl.BlockSpec((blk, 16), lambda i: (i, 0)),
                  pl.BlockSpec((blk, 32), lambda i: (i, 0)),
                  pl.BlockSpec((16, 32), lambda i: (0, 0)),
                  pl.BlockSpec((1, 32), lambda i: (0, 0)),
                  pl.BlockSpec((32, 16), lambda i: (0, 0)),
                  pl.BlockSpec((1, 16), lambda i: (0, 0))],
        out_specs=pl.BlockSpec((blk, 16), lambda i: (i, 0)),
        out_shape=jax.ShapeDtypeStruct((A_PAD, 16), jnp.float32),
    )(a12, g, wa, b1, w2, b2)


def _edge_body(s_ref, c_ref, e2_ref, w1a_ref, w1b_ref, b1_ref, w2_ref, b2_ref, o_ref):
    aggr = s_ref[...] / jnp.maximum(c_ref[...], 1.0)
    u = (jnp.dot(aggr, w1a_ref[...], preferred_element_type=jnp.float32)
         + jnp.dot(e2_ref[...], w1b_ref[...], preferred_element_type=jnp.float32)
         + b1_ref[...])
    h = _selu(u)
    o_ref[...] = jnp.dot(h, w2_ref[...], preferred_element_type=jnp.float32) + b2_ref[...]


def _edge_mlp(sums, cnt16, e2, w1a, w1b, b1, w2, b2, blk):
    n = e2.shape[0]
    return pl.pallas_call(
        _edge_body,
        grid=(n // blk,),
        in_specs=[pl.BlockSpec((blk, 16), lambda i: (i, 0)),
                  pl.BlockSpec((blk, 16), lambda i: (i, 0)),
                  pl.BlockSpec((blk, 64), lambda i: (i, 0)),
                  pl.BlockSpec((16, 32), lambda i: (0, 0)),
                  pl.BlockSpec((64, 32), lambda i: (0, 0)),
                  pl.BlockSpec((1, 32), lambda i: (0, 0)),
                  pl.BlockSpec((32, 64), lambda i: (0, 0)),
                  pl.BlockSpec((1, 64), lambda i: (0, 0))],
        out_specs=pl.BlockSpec((blk, 64), lambda i: (i, 0)),
        out_shape=jax.ShapeDtypeStruct((n, 64), jnp.float32),
    )(sums, cnt16, e2, w1a, w1b, b1, w2, b2)


_MESH = plsc.VectorSubcoreMesh(core_axis_name="c", subcore_axis_name="s")


@functools.partial(
    pl.kernel,
    out_type=jax.ShapeDtypeStruct((A_PAD, 32), jnp.bfloat16),
    mesh=_MESH,
    scratch_types=[
        pltpu.VMEM((2, GB, 128), jnp.int32),
        pltpu.VMEM((2, GB, 128), jnp.int32),
        pltpu.VMEM((2, CHB, 32), jnp.bfloat16),
        pltpu.VMEM((2, CHB, 32), jnp.bfloat16),
        pltpu.VMEM((2, CHB, 32), jnp.bfloat16),
        pltpu.SemaphoreType.DMA((2,)),
        pltpu.SemaphoreType.DMA((2,)),
    ],
    compiler_params=pltpu.CompilerParams(use_tc_tiling_on_sc=False),
)
def _gather_kernel(row_hbm, col_hbm, p1_hbm, p2_hbm, g_hbm, rowv, colv, b1, b2,
                   wbuf, gsem, wsem):
    c = lax.axis_index("c")
    s = lax.axis_index("s")
    wid = s * NC + c
    base = wid * PER_W

    def fire(i, p):
        off = pl.multiple_of(base + i * CHB, CHB)
        offr = pl.multiple_of(off // 128, GB)
        pltpu.sync_copy(row_hbm.at[pl.ds(offr, GB)], rowv.at[p])
        pltpu.sync_copy(col_hbm.at[pl.ds(offr, GB)], colv.at[p])
        for j in range(GB):
            pltpu.async_copy(p1_hbm.at[rowv.at[p, j]],
                             b1.at[p, pl.ds(j * 128, 128)], gsem.at[p])
            pltpu.async_copy(p2_hbm.at[colv.at[p, j]],
                             b2.at[p, pl.ds(j * 128, 128)], gsem.at[p])

    def drain_gathers(p):
        pltpu.make_async_copy(g_hbm.at[pl.ds(0, CHB)], b1.at[p], gsem.at[p]).wait()
        pltpu.make_async_copy(g_hbm.at[pl.ds(0, CHB)], b2.at[p], gsem.at[p]).wait()

    def drain_write(p):
        pltpu.make_async_copy(g_hbm.at[pl.ds(0, CHB)], wbuf.at[p], wsem.at[p]).wait()

    def consume(i, p):
        off = pl.multiple_of(base + i * CHB, CHB)
        drain_gathers(p)

        def add_row(r, _):
            wbuf[p, r, pl.ds(0, 32)] = b1[p, r, pl.ds(0, 32)] + b2[p, r, pl.ds(0, 32)]
            return ()

        lax.fori_loop(0, CHB, add_row, (), unroll=8)
        pltpu.async_copy(wbuf.at[p], g_hbm.at[pl.ds(off, CHB)], wsem.at[p])

    fire(0, 0)

    def pair(i2, _):
        i0 = i2 * 2
        fire(i0 + 1, 1)

        @pl.when(i2 > 0)
        def _():
            drain_write(0)
        consume(i0, 0)

        @pl.when(i2 < NCH_B // 2 - 1)
        def _():
            fire(i0 + 2, 0)

        @pl.when(i2 > 0)
        def _():
            drain_write(1)
        consume(i0 + 1, 1)
        return ()

    lax.fori_loop(0, NCH_B // 2, pair, ())
    drain_write(0)
    drain_write(1)


@functools.partial(
    pl.kernel,
    out_type=(jax.ShapeDtypeStruct((E2, 16), jnp.float32),
              jax.ShapeDtypeStruct((E2, 16), jnp.float32)),
    mesh=_MESH,
    scratch_types=[
        pltpu.VMEM((2, GD, 128), jnp.int32),
        pltpu.VMEM((2, GD, 128), jnp.int32),
        pltpu.VMEM((2, CHD, 16), jnp.float32),
        pltpu.VMEM((128, 16), jnp.float32),
        pltpu.VMEM_SHARED((SROWS, 16), jnp.float32),
        pltpu.SemaphoreType.DMA((2,)),
        pltpu.SemaphoreType.DMA((2,)),
    ],
    compiler_params=pltpu.CompilerParams(use_tc_tiling_on_sc=False),
)
def _scatter_kernel(col_hbm, a_hbm, sums_hbm, cnt_hbm, colv, sidxv, abuf, cbuf,
                    S, lsem, ssem):
    c = lax.axis_index("c")
    s = lax.axis_index("s")
    base = s * PER_S
    seg0 = c * E2H

    def fill_cbuf(val):
        def fb(i, _):
            cbuf[i, pl.ds(0, 16)] = jnp.full((16,), val, jnp.float32)
            return ()
        lax.fori_loop(0, 128, fb, (), unroll=8)

    def zero_stripe(r0, total):
        done = 0
        while done < total:
            nrows = min(128, total - done)
            pltpu.sync_copy(cbuf.at[pl.ds(0, nrows)],
                            S.at[pl.ds(pl.multiple_of(r0 + done, 8), nrows)])
            done += nrows

    def zero_s():
        @pl.when(s < 15)
        def _():
            zero_stripe(s * STRIPE, STRIPE)

        @pl.when(s == 15)
        def _():
            zero_stripe(15 * STRIPE, ZLAST)

    def fire(i, p, with_a):
        off = pl.multiple_of(base + i * CHD, CHD)
        offr = pl.multiple_of(off // 128, GD)
        pltpu.async_copy(col_hbm.at[pl.ds(offr, GD)], colv.at[p], lsem.at[p])
        if with_a:
            pltpu.async_copy(a_hbm.at[pl.ds(off, CHD)], abuf.at[p], lsem.at[p])

    def drain_load(p, with_a):
        pltpu.make_async_copy(col_hbm.at[pl.ds(0, GD)], colv.at[p],
                              lsem.at[p]).wait()
        if with_a:
            pltpu.make_async_copy(a_hbm.at[pl.ds(0, CHD)], abuf.at[p],
                                  lsem.at[p]).wait()

    def drain_scatter(p):
        pltpu.make_async_copy(a_hbm.at[pl.ds(0, CHD)], abuf.at[p],
                              ssem.at[p]).wait()

    def consume(i, p, with_a):
        off = pl.multiple_of(base + i * CHD, CHD)
        drain_load(p, with_a)

        def vb(j, _):
            jj = j // 8
            kk = j - jj * 8
            cv = colv[p, jj, pl.ds(kk * 16, 16)]
            gidx = off + j * 16 + lax.iota(jnp.int32, 16)
            lcol = cv - seg0
            valid = (lcol >= 0) & (lcol < E2H) & (gidx < A)
            sidxv[p, jj, pl.ds(kk * 16, 16)] = jnp.where(valid, lcol, E2H)
            return ()

        lax.fori_loop(0, CHD // 16, vb, (), unroll=4)
        for j in range(GD):
            src = abuf.at[p, pl.ds(j * 128, 128)] if with_a else cbuf
            pltpu.async_copy(src, S.at[sidxv.at[p, j]], ssem.at[p], add=True)

    def sweep(with_a):
        fire(0, 0, with_a)

        def pair(i2, _):
            i0 = i2 * 2

            @pl.when(i2 > 0)
            def _():
                drain_scatter(1)
            fire(i0 + 1, 1, with_a)
            consume(i0, 0, with_a)
            consume(i0 + 1, 1, with_a)

            @pl.when(i2 < NCHD // 2 - 1)
            def _():
                drain_scatter(0)
                fire(i0 + 2, 0, with_a)
            return ()

        lax.fori_loop(0, NCHD // 2, pair, ())
        drain_scatter(0)
        drain_scatter(1)

    def flush(dst_hbm):
        @pl.when(s < 15)
        def _():
            r0 = pl.multiple_of(s * STRIPE, 8)
            pltpu.sync_copy(S.at[pl.ds(r0, STRIPE)],
                            dst_hbm.at[pl.ds(pl.multiple_of(seg0 + r0, 8), STRIPE)])

        @pl.when(s == 15)
        def _():
            r0 = 15 * STRIPE
            pltpu.sync_copy(S.at[pl.ds(r0, FLAST)],
                            dst_hbm.at[pl.ds(pl.multiple_of(seg0 + r0, 8), FLAST)])

    fill_cbuf(0.0)
    zero_s()
    plsc.subcore_barrier()
    sweep(True)
    plsc.subcore_barrier()
    flush(sums_hbm)
    plsc.subcore_barrier()
    zero_s()
    fill_cbuf(1.0)
    plsc.subcore_barrier()
    sweep(False)
    plsc.subcore_barrier()
    flush(cnt_hbm)


def kernel(e1, e2, a12, angle_index12, aw1, ab1, aw2, ab2, ew1, eb1, ew2, eb2):
    w_a = aw1[:16]
    w_e1 = aw1[16:80]
    w_e2 = aw1[80:144]

    p1 = _rows_mm(e1, w_e1, 4000, jnp.bfloat16)
    p2 = _rows_mm(e2, w_e2, 4000, jnp.bfloat16)

    pad = A_PAD - A
    row2 = jnp.pad(angle_index12[0], (0, pad)).reshape(A_PAD // 128, 128)
    col2 = jnp.pad(angle_index12[1], (0, pad)).reshape(A_PAD // 128, 128)

    g = _gather_kernel(row2, col2, p1, p2)

    a = _angle_mlp(a12, g, w_a, ab1.reshape(1, 32), aw2, ab2.reshape(1, 16), 4000)

    sums, cnt16 = _scatter_kernel(col2, a)

    out = _edge_mlp(sums, cnt16, e2, ew1[:16], ew1[16:], eb1.reshape(1, 32),
                    ew2, eb2.reshape(1, 64), 2000)
    return out
